# baseline (speedup 1.0000x reference)
import functools
import math

import jax
import jax.numpy as jnp
from jax import lax
from jax.experimental import pallas as pl
from jax.experimental.pallas import tpu as pltpu
from jax.experimental.pallas import tpu_sc as plsc

NC = 2
NS = 16
NW = NC * NS
LANES = 16
CH = 128


def _split(tot, frac0):
    n0 = int(round(tot * frac0 / 2)) * 2
    n0 = min(max(n0, 2), tot - 2)
    return n0, tot - n0


def SPLIT_L1(tot):
    return _split(tot, 0.5)


def SPLIT_L2(tot):
    return _split(tot, 0.565)


def _mesh():
    return plsc.VectorSubcoreMesh(core_axis_name="c", subcore_axis_name="s")


def _sc_kernel(out_shapes, scratch):
    return functools.partial(
        pl.kernel,
        out_type=tuple(jax.ShapeDtypeStruct(s, jnp.float32)
                       for s in out_shapes),
        mesh=_mesh(),
        compiler_params=pltpu.CompilerParams(use_tc_tiling_on_sc=False),
        scratch_types=scratch,
    )


def _copy_idx(srcall, dstall, srcv, dstv, cur):
    for g in range(CH // LANES):
        sl = pl.ds(cur * CH + g * LANES, LANES)
        so = pl.ds(g * LANES, LANES)
        srcv[so] = srcall[sl]
        dstv[so] = dstall[sl]


def _scatter_idx(srcv, dstv, sidx, trash):
    for g in range(CH // LANES):
        sl = pl.ds(g * LANES, LANES)
        sv = srcv[sl]
        dv = dstv[sl]
        sidx[sl] = jnp.where(sv != dv, dv, trash)


def _sc_sweep_l1(n_pad, nch0, nch1):
    ept = max(nch0, nch1) * CH

    @_sc_kernel([(NC, n_pad, 80)], [
        pltpu.VMEM((ept,), jnp.int32),
        pltpu.VMEM((ept,), jnp.int32),
        pltpu.VMEM((CH,), jnp.int32),
        pltpu.VMEM((CH,), jnp.int32),
        pltpu.VMEM((CH,), jnp.int32),
        pltpu.VMEM((CH,), jnp.int32),
        pltpu.VMEM((CH,), jnp.int32),
        pltpu.VMEM((CH,), jnp.int32),
        pltpu.VMEM((CH, 80), jnp.float32),
        pltpu.VMEM((CH, 80), jnp.float32),
        pltpu.VMEM((CH, 16), jnp.float32),
        pltpu.VMEM((CH, 16), jnp.float32),
        pltpu.VMEM_SHARED((n_pad, 80), jnp.float32),
        pltpu.SemaphoreType.DMA,
        pltpu.SemaphoreType.DMA,
        pltpu.SemaphoreType.DMA,
        pltpu.SemaphoreType.DMA,
    ])
    def k(srcp, dstp, tsh, td, z80, outa, srcall, dstall,
          srcv0, srcv1, dstv0, dstv1, sidx0, sidx1, sh0, sh1, rd0, rd1,
          acc, gs0, gs1, ss0, ss1):
        c = lax.axis_index("c")
        s = lax.axis_index("s")
        trash = n_pad - 8
        lane = lax.iota(jnp.int32, LANES)
        head_idx = [2 * k2 + jnp.right_shift(lane, 3) for k2 in range(4)]
        srcv_ = [srcv0, srcv1]
        dstv_ = [dstv0, dstv1]
        sh_ = [sh0, sh1]
        rd_ = [rd0, rd1]
        six_ = [sidx0, sidx1]
        gs_ = [gs0, gs1]
        ss_ = [ss0, ss1]

        @pl.when(s == 0)
        def _init():
            pltpu.sync_copy(z80, acc)

        nch_local = jnp.where(c == 0, nch0, nch1)
        base = pl.multiple_of(
            jnp.where(c == 0, s * nch0, 16 * nch0 + s * nch1) * CH, CH)
        pltpu.sync_copy(srcp.at[pl.ds(base, ept)], srcall)
        pltpu.sync_copy(dstp.at[pl.ds(base, ept)], dstall)
        plsc.subcore_barrier()

        def gathers(b):
            return (
                pltpu.make_async_copy(tsh.at[srcv_[b]], sh_[b], gs_[b]),
                pltpu.make_async_copy(td.at[dstv_[b]], rd_[b], gs_[b]),
            )

        def issue(cur, b):
            _copy_idx(srcall, dstall, srcv_[b], dstv_[b], cur)
            for d in gathers(b):
                d.start()

        def scatter_start(b):
            pltpu.async_copy(sh_[b], acc.at[six_[b]], ss_[b], add=True)

        def scatter_wait(b):
            pltpu.make_async_copy(sh_[b], acc.at[six_[b]], ss_[b]).wait()

        issue(0, 0)
        issue(1, 1)

        def gg_body(gg, _):
            for b in range(2):
                cur = 2 * gg + b
                for d in gathers(b):
                    d.wait()
                _scatter_idx(srcv_[b], dstv_[b], six_[b], trash)
                for i in range(CH):
                    e = sh_[b][i, pl.ds(0, LANES)] + rd_[b][i]
                    lr = jnp.where(e > 0.0, e, e * 0.2)
                    ex = jnp.exp(lr)
                    sh_[b][i, pl.ds(0, LANES)] = ex
                    for k2 in range(4):
                        hsl = pl.ds(16 + k2 * LANES, LANES)
                        av = jnp.take_along_axis(ex, head_idx[k2], axis=0)
                        sh_[b][i, hsl] = sh_[b][i, hsl] * av
                scatter_start(b)

                @pl.when(cur + 2 < nch_local)
                def _next():
                    scatter_wait(b)
                    issue(cur + 2, b)
            return ()

        lax.fori_loop(0, nch_local // 2, gg_body, ())
        scatter_wait(0)
        scatter_wait(1)
        plsc.subcore_barrier()

        @pl.when(s == 0)
        def _flush():
            pltpu.sync_copy(acc, outa.at[c])

    return k


def _sc_sweep_l2(n_pad, nch0, nch1):
    ept = max(nch0, nch1) * CH

    @_sc_kernel([(NC, n_pad), (NC, n_pad, 48)], [
        pltpu.VMEM((ept,), jnp.int32),
        pltpu.VMEM((ept,), jnp.int32),
        pltpu.VMEM((CH,), jnp.int32),
        pltpu.VMEM((CH,), jnp.int32),
        pltpu.VMEM((CH,), jnp.int32),
        pltpu.VMEM((CH,), jnp.int32),
        pltpu.VMEM((CH,), jnp.int32),
        pltpu.VMEM((CH,), jnp.int32),
        pltpu.VMEM((CH,), jnp.float32),
        pltpu.VMEM((CH,), jnp.float32),
        pltpu.VMEM((CH,), jnp.float32),
        pltpu.VMEM((CH,), jnp.float32),
        pltpu.VMEM((CH, 48), jnp.float32),
        pltpu.VMEM((CH, 48), jnp.float32),
        pltpu.VMEM_SHARED((n_pad,), jnp.float32),
        pltpu.VMEM_SHARED((n_pad, 48), jnp.float32),
        pltpu.SemaphoreType.DMA,
        pltpu.SemaphoreType.DMA,
        pltpu.SemaphoreType.DMA,
        pltpu.SemaphoreType.DMA,
    ])
    def k(srcp, dstp, asrc, adst, h2, z1, z48, outs, outo, srcall, dstall,
          srcv0, srcv1, dstv0, dstv1, sidx0, sidx1, av0, av1, bv0, bv1,
          hr0, hr1, sacc, oacc, gs0, gs1, ss0, ss1):
        c = lax.axis_index("c")
        s = lax.axis_index("s")
        wid = c * NS + s
        trash = n_pad - 8
        lane = lax.iota(jnp.int32, LANES)
        srcv_ = [srcv0, srcv1]
        dstv_ = [dstv0, dstv1]
        av_ = [av0, av1]
        bv_ = [bv0, bv1]
        hr_ = [hr0, hr1]
        six_ = [sidx0, sidx1]
        gs_ = [gs0, gs1]
        ss_ = [ss0, ss1]

        @pl.when(s == 0)
        def _init():
            pltpu.sync_copy(z1, sacc)
            pltpu.sync_copy(z48, oacc)

        nch_local = jnp.where(c == 0, nch0, nch1)
        base = pl.multiple_of(
            jnp.where(c == 0, s * nch0, 16 * nch0 + s * nch1) * CH, CH)
        pltpu.sync_copy(srcp.at[pl.ds(base, ept)], srcall)
        pltpu.sync_copy(dstp.at[pl.ds(base, ept)], dstall)
        plsc.subcore_barrier()

        def gathers(b):
            return (
                pltpu.make_async_copy(asrc.at[srcv_[b]], av_[b], gs_[b]),
                pltpu.make_async_copy(adst.at[dstv_[b]], bv_[b], gs_[b]),
                pltpu.make_async_copy(h2.at[srcv_[b]], hr_[b], gs_[b]),
            )

        def issue(cur, b):
            _copy_idx(srcall, dstall, srcv_[b], dstv_[b], cur)
            for d in gathers(b):
                d.start()

        def scatter_start(b):
            pltpu.async_copy(av_[b], sacc.at[six_[b]], ss_[b], add=True)
            pltpu.async_copy(hr_[b], oacc.at[six_[b]], ss_[b], add=True)

        def scatter_wait(b):
            pltpu.make_async_copy(av_[b], sacc.at[six_[b]], ss_[b]).wait()
            pltpu.make_async_copy(hr_[b], oacc.at[six_[b]], ss_[b]).wait()

        issue(0, 0)
        issue(1, 1)

        def gg_body(gg, _):
            for b in range(2):
                cur = 2 * gg + b
                for d in gathers(b):
                    d.wait()
                _scatter_idx(srcv_[b], dstv_[b], six_[b], trash)
                for g in range(CH // LANES):
                    sl = pl.ds(g * LANES, LANES)
                    e = av_[b][sl] + bv_[b][sl]
                    lr = jnp.where(e > 0.0, e, e * 0.2)
                    av_[b][sl] = jnp.exp(lr)
                for g in range(CH // LANES):
                    a16 = av_[b][pl.ds(g * LANES, LANES)]
                    for j in range(LANES):
                        i = g * LANES + j
                        asp = jnp.take_along_axis(a16, lane * 0 + j, axis=0)
                        for k2 in range(3):
                            hsl = pl.ds(k2 * LANES, LANES)
                            hr_[b][i, hsl] = hr_[b][i, hsl] * asp
                scatter_start(b)

                @pl.when(cur + 2 < nch_local)
                def _next():
                    scatter_wait(b)
                    issue(cur + 2, b)
            return ()

        lax.fori_loop(0, nch_local // 2, gg_body, ())
        scatter_wait(0)
        scatter_wait(1)
        plsc.subcore_barrier()

        @pl.when(s == 0)
        def _flush():
            pltpu.sync_copy(sacc, outs.at[c])
            pltpu.sync_copy(oacc, outo.at[c])

    return k


def _tc1(x, w1, ms1, md1, n, rblk):
    def body(x_ref, w_ref, ms_ref, md_ref, tsh_ref, td_ref):
        h = jnp.dot(x_ref[...], w_ref[...], preferred_element_type=jnp.float32)
        ts = jnp.dot(h, ms_ref[...], preferred_element_type=jnp.float32)
        tsh_ref[...] = jnp.concatenate([ts, h], axis=1)
        td_ref[...] = jnp.dot(h, md_ref[...], preferred_element_type=jnp.float32)

    g = n // rblk
    return pl.pallas_call(
        body,
        grid=(g,),
        in_specs=[
            pl.BlockSpec((rblk, 128), lambda i: (i, 0)),
            pl.BlockSpec((128, 64), lambda i: (0, 0)),
            pl.BlockSpec((64, 16), lambda i: (0, 0)),
            pl.BlockSpec((64, 16), lambda i: (0, 0)),
        ],
        out_specs=[
            pl.BlockSpec((rblk, 80), lambda i: (i, 0)),
            pl.BlockSpec((rblk, 16), lambda i: (i, 0)),
        ],
        out_shape=[
            jax.ShapeDtypeStruct((n, 80), jnp.float32),
            jax.ShapeDtypeStruct((n, 16), jnp.float32),
        ],
    )(x, w1, ms1, md1)


def _tc2(aparts, tsh1, td1, b1, w2, m2, r8, n, rblk):
    def body(ap_ref, tsh_ref, td_ref, b1_ref, w2_ref, m2_ref, r8_ref,
             h2p_ref, att2_ref):
        ap = ap_ref[...]
        tsh = tsh_ref[...]
        h1 = tsh[:, 16:80]
        e = tsh[:, 0:8] + td_ref[...][:, 0:8]
        ex = jnp.exp(jnp.where(e > 0.0, e, e * 0.2))
        stot = ap[0][:, 0:8] + ap[1][:, 0:8] + ex
        rs = 1.0 / (stot + 1e-16)
        rs_rep = jnp.dot(rs, r8_ref[...], preferred_element_type=jnp.float32)
        aself_rep = jnp.dot(ex * rs, r8_ref[...],
                            preferred_element_type=jnp.float32)
        o = ((ap[0][:, 16:80] + ap[1][:, 16:80]) * rs_rep + aself_rep * h1
             + b1_ref[...])
        eo = jnp.where(o > 0.0, o, jnp.exp(o) - 1.0)
        h2 = jnp.dot(eo, w2_ref[...], preferred_element_type=jnp.float32)
        h2p_ref[...] = jnp.concatenate(
            [h2, jnp.zeros((h2.shape[0], 8), jnp.float32)], axis=1)
        att2_ref[...] = jnp.dot(h2, m2_ref[...],
                                preferred_element_type=jnp.float32)

    g = n // rblk
    return pl.pallas_call(
        body,
        grid=(g,),
        in_specs=[
            pl.BlockSpec((2, rblk, 80), lambda i: (0, i, 0)),
            pl.BlockSpec((rblk, 80), lambda i: (i, 0)),
            pl.BlockSpec((rblk, 16), lambda i: (i, 0)),
            pl.BlockSpec((1, 64), lambda i: (0, 0)),
            pl.BlockSpec((64, 40), lambda i: (0, 0)),
            pl.BlockSpec((40, 8), lambda i: (0, 0)),
            pl.BlockSpec((8, 64), lambda i: (0, 0)),
        ],
        out_specs=[
            pl.BlockSpec((rblk, 48), lambda i: (i, 0)),
            pl.BlockSpec((rblk, 8), lambda i: (i, 0)),
        ],
        out_shape=[
            jax.ShapeDtypeStruct((n, 48), jnp.float32),
            jax.ShapeDtypeStruct((n, 8), jnp.float32),
        ],
    )(aparts, tsh1, td1, b1, w2, m2, r8)


def _tc3(o2parts, s2parts, att2, h2p, b2, n, rblk):
    def body(op_ref, sp_ref, att_ref, h2_ref, b2_ref, out_ref):
        op = op_ref[...]
        att = att_ref[...]
        e = att[:, 0:1] + att[:, 1:2]
        ex = jnp.exp(jnp.where(e > 0.0, e, e * 0.2))
        sp = sp_ref[...]
        stot = sp[:, 0:1] + sp[:, 1:2] + ex
        rs = 1.0 / (stot + 1e-16)
        o = ((op[0][:, 0:40] + op[1][:, 0:40]) * rs
             + (ex * rs) * h2_ref[...][:, 0:40] + b2_ref[...])
        m = jnp.max(o, axis=1, keepdims=True)
        lse = jnp.log(jnp.sum(jnp.exp(o - m), axis=1, keepdims=True)) + m
        out_ref[...] = o - lse

    g = n // rblk
    return pl.pallas_call(
        body,
        grid=(g,),
        in_specs=[
            pl.BlockSpec((2, rblk, 48), lambda i: (0, i, 0)),
            pl.BlockSpec((rblk, 2), lambda i: (i, 0)),
            pl.BlockSpec((rblk, 8), lambda i: (i, 0)),
            pl.BlockSpec((rblk, 48), lambda i: (i, 0)),
            pl.BlockSpec((1, 40), lambda i: (0, 0)),
        ],
        out_specs=pl.BlockSpec((rblk, 40), lambda i: (i, 0)),
        out_shape=jax.ShapeDtypeStruct((n, 40), jnp.float32),
    )(o2parts, s2parts, att2, h2p, b2)


def kernel(x, edge_index, W1, att_src1, att_dst1, b1, W2, att_src2, att_dst2,
           b2):
    n, _ = x.shape
    e_num = edge_index.shape[1]
    h1dim = W1.shape[1]
    c2 = W2.shape[1]
    n_pad = n + 8
    rblk = min(2000, n)

    tot = 2 * math.ceil(e_num / (NW * CH))
    tot += tot % 2
    nch0_l1, nch1_l1 = SPLIT_L1(tot)
    nch0_l2, nch1_l2 = SPLIT_L2(tot)
    max_nch = max(nch0_l1, nch1_l1, nch0_l2, nch1_l2)
    e_pad = NS * tot * CH + max_nch * CH
    pad = e_pad - e_num
    src = jnp.concatenate([edge_index[0], jnp.zeros((pad,), jnp.int32)])
    dst = jnp.concatenate([edge_index[1], jnp.zeros((pad,), jnp.int32)])

    hc = jnp.arange(h1dim)
    ms1 = jnp.zeros((h1dim, 16), jnp.float32).at[hc, hc // 8].set(
        att_src1.reshape(h1dim))
    md1 = jnp.zeros((h1dim, 16), jnp.float32).at[hc, hc // 8].set(
        att_dst1.reshape(h1dim))
    r8 = jnp.zeros((8, h1dim), jnp.float32).at[hc // 8, hc].set(1.0)
    m2 = jnp.zeros((c2, 8), jnp.float32)
    m2 = m2.at[:, 0].set(att_src2.reshape(c2)).at[:, 1].set(
        att_dst2.reshape(c2))

    z80 = jnp.zeros((n_pad, 80), jnp.float32)
    z48 = jnp.zeros((n_pad, 48), jnp.float32)
    z1 = jnp.zeros((n_pad,), jnp.float32)

    tsh1, td1 = _tc1(x, W1, ms1, md1, n, rblk)
    (aparts,) = _sc_sweep_l1(n_pad, nch0_l1, nch1_l1)(
        src, dst, tsh1, td1, z80)
    h2p, att2 = _tc2(aparts[:, :n, :], tsh1, td1,
                     b1.reshape(1, -1), W2, m2, r8, n, rblk)

    s2parts, o2parts = _sc_sweep_l2(n_pad, nch0_l2, nch1_l2)(
        src, dst, att2[:, 0], att2[:, 1], h2p, z1, z48)
    return _tc3(o2parts[:, :n, :], s2parts[:, :n].T, att2, h2p,
                b2.reshape(1, -1), n, rblk)


# speedup vs baseline: 1.0907x; 1.0907x over previous
import functools
import math

import jax
import jax.numpy as jnp
from jax import lax
from jax.experimental import pallas as pl
from jax.experimental.pallas import tpu as pltpu
from jax.experimental.pallas import tpu_sc as plsc

NC = 2
NS = 16
NW = NC * NS
LANES = 16
CH = 128


def _split(tot, frac0):
    n0 = int(round(tot * frac0 / 2)) * 2
    n0 = min(max(n0, 2), tot - 2)
    return n0, tot - n0


def SPLIT_L1(tot):
    return _split(tot, 0.5)


def SPLIT_L2(tot):
    return _split(tot, 0.565)


def _mesh():
    return plsc.VectorSubcoreMesh(core_axis_name="c", subcore_axis_name="s")


def _sc_kernel(out_shapes, scratch):
    return functools.partial(
        pl.kernel,
        out_type=tuple(jax.ShapeDtypeStruct(s, jnp.float32)
                       for s in out_shapes),
        mesh=_mesh(),
        compiler_params=pltpu.CompilerParams(use_tc_tiling_on_sc=False),
        scratch_types=scratch,
    )


def _copy_idx(srcall, dstall, srcv, dstv, cur):
    for g in range(CH // LANES):
        sl = pl.ds(cur * CH + g * LANES, LANES)
        so = pl.ds(g * LANES, LANES)
        srcv[so] = srcall[sl]
        dstv[so] = dstall[sl]


def _scatter_idx(srcv, dstv, sidx, trash):
    for g in range(CH // LANES):
        sl = pl.ds(g * LANES, LANES)
        sv = srcv[sl]
        dv = dstv[sl]
        sidx[sl] = jnp.where(sv != dv, dv, trash)


def _sc_sweep_l1(n_pad, nch0, nch1):
    ept = max(nch0, nch1) * CH

    @_sc_kernel([(NC, n_pad, 16), (NC, n_pad, 64)], [
        pltpu.VMEM((ept,), jnp.int32),
        pltpu.VMEM((ept,), jnp.int32),
        pltpu.VMEM((CH,), jnp.int32),
        pltpu.VMEM((CH,), jnp.int32),
        pltpu.VMEM((CH,), jnp.int32),
        pltpu.VMEM((CH,), jnp.int32),
        pltpu.VMEM((CH,), jnp.int32),
        pltpu.VMEM((CH,), jnp.int32),
        pltpu.VMEM((CH, 16), jnp.float32),
        pltpu.VMEM((CH, 16), jnp.float32),
        pltpu.VMEM((CH, 16), jnp.float32),
        pltpu.VMEM((CH, 16), jnp.float32),
        pltpu.VMEM((CH, 64), jnp.float32),
        pltpu.VMEM((CH, 64), jnp.float32),
        pltpu.VMEM_SHARED((n_pad, 16), jnp.float32),
        pltpu.VMEM_SHARED((n_pad, 64), jnp.float32),
        pltpu.SemaphoreType.DMA,
        pltpu.SemaphoreType.DMA,
        pltpu.SemaphoreType.DMA,
        pltpu.SemaphoreType.DMA,
    ])
    def k(srcp, dstp, ts, td, h1, z16, z64, outs, outo, srcall, dstall,
          srcv0, srcv1, dstv0, dstv1, sidx0, sidx1, rs0, rs1, rd0, rd1,
          hr0, hr1, sacc, oacc, gs0, gs1, ss0, ss1):
        c = lax.axis_index("c")
        s = lax.axis_index("s")
        wid = c * NS + s
        trash = n_pad - 8
        lane = lax.iota(jnp.int32, LANES)
        head_idx = [2 * k2 + jnp.right_shift(lane, 3) for k2 in range(4)]
        srcv_ = [srcv0, srcv1]
        dstv_ = [dstv0, dstv1]
        rs_ = [rs0, rs1]
        rd_ = [rd0, rd1]
        hr_ = [hr0, hr1]
        six_ = [sidx0, sidx1]
        gs_ = [gs0, gs1]
        ss_ = [ss0, ss1]

        @pl.when(s == 0)
        def _init():
            pltpu.sync_copy(z16, sacc)
            pltpu.sync_copy(z64, oacc)

        nch_local = jnp.where(c == 0, nch0, nch1)
        base = pl.multiple_of(
            jnp.where(c == 0, s * nch0, 16 * nch0 + s * nch1) * CH, CH)
        pltpu.sync_copy(srcp.at[pl.ds(base, ept)], srcall)
        pltpu.sync_copy(dstp.at[pl.ds(base, ept)], dstall)
        plsc.subcore_barrier()

        def gathers(b):
            return (
                pltpu.make_async_copy(ts.at[srcv_[b]], rs_[b], gs_[b]),
                pltpu.make_async_copy(td.at[dstv_[b]], rd_[b], gs_[b]),
                pltpu.make_async_copy(h1.at[srcv_[b]], hr_[b], gs_[b]),
            )

        def issue(cur, b):
            _copy_idx(srcall, dstall, srcv_[b], dstv_[b], cur)
            for d in gathers(b):
                d.start()

        def scatter_start(b):
            pltpu.async_copy(rs_[b], sacc.at[six_[b]], ss_[b], add=True)
            pltpu.async_copy(hr_[b], oacc.at[six_[b]], ss_[b], add=True)

        def scatter_wait(b):
            pltpu.make_async_copy(rs_[b], sacc.at[six_[b]], ss_[b]).wait()
            pltpu.make_async_copy(hr_[b], oacc.at[six_[b]], ss_[b]).wait()

        issue(0, 0)
        issue(1, 1)

        def gg_body(gg, _):
            for b in range(2):
                cur = 2 * gg + b
                for d in gathers(b):
                    d.wait()
                _scatter_idx(srcv_[b], dstv_[b], six_[b], trash)
                for i in range(CH):
                    e = rs_[b][i] + rd_[b][i]
                    lr = jnp.where(e > 0.0, e, e * 0.2)
                    ex = jnp.exp(lr)
                    rs_[b][i] = ex
                    for k2 in range(4):
                        hsl = pl.ds(k2 * LANES, LANES)
                        av = jnp.take_along_axis(ex, head_idx[k2], axis=0)
                        hr_[b][i, hsl] = hr_[b][i, hsl] * av
                scatter_start(b)

                @pl.when(cur + 2 < nch_local)
                def _next():
                    scatter_wait(b)
                    issue(cur + 2, b)
            return ()

        lax.fori_loop(0, nch_local // 2, gg_body, ())
        scatter_wait(0)
        scatter_wait(1)
        plsc.subcore_barrier()

        @pl.when(s == 0)
        def _flush():
            pltpu.sync_copy(sacc, outs.at[c])
            pltpu.sync_copy(oacc, outo.at[c])

    return k


def _sc_sweep_l2(n_pad, nch0, nch1):
    ept = max(nch0, nch1) * CH

    @_sc_kernel([(NC, n_pad), (NC, n_pad, 48)], [
        pltpu.VMEM((ept,), jnp.int32),
        pltpu.VMEM((ept,), jnp.int32),
        pltpu.VMEM((CH,), jnp.int32),
        pltpu.VMEM((CH,), jnp.int32),
        pltpu.VMEM((CH,), jnp.int32),
        pltpu.VMEM((CH,), jnp.int32),
        pltpu.VMEM((CH,), jnp.int32),
        pltpu.VMEM((CH,), jnp.int32),
        pltpu.VMEM((CH,), jnp.float32),
        pltpu.VMEM((CH,), jnp.float32),
        pltpu.VMEM((CH,), jnp.float32),
        pltpu.VMEM((CH,), jnp.float32),
        pltpu.VMEM((CH, 48), jnp.float32),
        pltpu.VMEM((CH, 48), jnp.float32),
        pltpu.VMEM_SHARED((n_pad,), jnp.float32),
        pltpu.VMEM_SHARED((n_pad, 48), jnp.float32),
        pltpu.SemaphoreType.DMA,
        pltpu.SemaphoreType.DMA,
        pltpu.SemaphoreType.DMA,
        pltpu.SemaphoreType.DMA,
    ])
    def k(srcp, dstp, asrc, adst, h2, z1, z48, outs, outo, srcall, dstall,
          srcv0, srcv1, dstv0, dstv1, sidx0, sidx1, av0, av1, bv0, bv1,
          hr0, hr1, sacc, oacc, gs0, gs1, ss0, ss1):
        c = lax.axis_index("c")
        s = lax.axis_index("s")
        wid = c * NS + s
        trash = n_pad - 8
        lane = lax.iota(jnp.int32, LANES)
        srcv_ = [srcv0, srcv1]
        dstv_ = [dstv0, dstv1]
        av_ = [av0, av1]
        bv_ = [bv0, bv1]
        hr_ = [hr0, hr1]
        six_ = [sidx0, sidx1]
        gs_ = [gs0, gs1]
        ss_ = [ss0, ss1]

        @pl.when(s == 0)
        def _init():
            pltpu.sync_copy(z1, sacc)
            pltpu.sync_copy(z48, oacc)

        nch_local = jnp.where(c == 0, nch0, nch1)
        base = pl.multiple_of(
            jnp.where(c == 0, s * nch0, 16 * nch0 + s * nch1) * CH, CH)
        pltpu.sync_copy(srcp.at[pl.ds(base, ept)], srcall)
        pltpu.sync_copy(dstp.at[pl.ds(base, ept)], dstall)
        plsc.subcore_barrier()

        def gathers(b):
            return (
                pltpu.make_async_copy(asrc.at[srcv_[b]], av_[b], gs_[b]),
                pltpu.make_async_copy(adst.at[dstv_[b]], bv_[b], gs_[b]),
                pltpu.make_async_copy(h2.at[srcv_[b]], hr_[b], gs_[b]),
            )

        def issue(cur, b):
            _copy_idx(srcall, dstall, srcv_[b], dstv_[b], cur)
            for d in gathers(b):
                d.start()

        def scatter_start(b):
            pltpu.async_copy(av_[b], sacc.at[six_[b]], ss_[b], add=True)
            pltpu.async_copy(hr_[b], oacc.at[six_[b]], ss_[b], add=True)

        def scatter_wait(b):
            pltpu.make_async_copy(av_[b], sacc.at[six_[b]], ss_[b]).wait()
            pltpu.make_async_copy(hr_[b], oacc.at[six_[b]], ss_[b]).wait()

        issue(0, 0)
        issue(1, 1)

        def gg_body(gg, _):
            for b in range(2):
                cur = 2 * gg + b
                for d in gathers(b):
                    d.wait()
                _scatter_idx(srcv_[b], dstv_[b], six_[b], trash)
                for g in range(CH // LANES):
                    sl = pl.ds(g * LANES, LANES)
                    e = av_[b][sl] + bv_[b][sl]
                    lr = jnp.where(e > 0.0, e, e * 0.2)
                    av_[b][sl] = jnp.exp(lr)
                for g in range(CH // LANES):
                    a16 = av_[b][pl.ds(g * LANES, LANES)]
                    for j in range(LANES):
                        i = g * LANES + j
                        asp = jnp.take_along_axis(a16, lane * 0 + j, axis=0)
                        for k2 in range(3):
                            hsl = pl.ds(k2 * LANES, LANES)
                            hr_[b][i, hsl] = hr_[b][i, hsl] * asp
                scatter_start(b)

                @pl.when(cur + 2 < nch_local)
                def _next():
                    scatter_wait(b)
                    issue(cur + 2, b)
            return ()

        lax.fori_loop(0, nch_local // 2, gg_body, ())
        scatter_wait(0)
        scatter_wait(1)
        plsc.subcore_barrier()

        @pl.when(s == 0)
        def _flush():
            pltpu.sync_copy(sacc, outs.at[c])
            pltpu.sync_copy(oacc, outo.at[c])

    return k


def _tc1(x, w1, ms1, md1, n, rblk):
    def body(x_ref, w_ref, ms_ref, md_ref, h_ref, ts_ref, td_ref):
        h = jnp.dot(x_ref[...], w_ref[...], preferred_element_type=jnp.float32)
        h_ref[...] = h
        ts_ref[...] = jnp.dot(h, ms_ref[...], preferred_element_type=jnp.float32)
        td_ref[...] = jnp.dot(h, md_ref[...], preferred_element_type=jnp.float32)

    g = n // rblk
    return pl.pallas_call(
        body,
        grid=(g,),
        in_specs=[
            pl.BlockSpec((rblk, 128), lambda i: (i, 0)),
            pl.BlockSpec((128, 64), lambda i: (0, 0)),
            pl.BlockSpec((64, 16), lambda i: (0, 0)),
            pl.BlockSpec((64, 16), lambda i: (0, 0)),
        ],
        out_specs=[
            pl.BlockSpec((rblk, 64), lambda i: (i, 0)),
            pl.BlockSpec((rblk, 16), lambda i: (i, 0)),
            pl.BlockSpec((rblk, 16), lambda i: (i, 0)),
        ],
        out_shape=[
            jax.ShapeDtypeStruct((n, 64), jnp.float32),
            jax.ShapeDtypeStruct((n, 16), jnp.float32),
            jax.ShapeDtypeStruct((n, 16), jnp.float32),
        ],
    )(x, w1, ms1, md1)


def _tc2(sparts, oparts, ts1, td1, h1, b1, w2, m2, r8, n, rblk):
    def body(sp_ref, op_ref, ts_ref, td_ref, h1_ref, b1_ref, w2_ref,
             m2_ref, r8_ref, h2p_ref, att2_ref):
        sp = sp_ref[...]
        op = op_ref[...]
        e = ts_ref[...][:, 0:8] + td_ref[...][:, 0:8]
        ex = jnp.exp(jnp.where(e > 0.0, e, e * 0.2))
        stot = sp[0][:, 0:8] + sp[1][:, 0:8] + ex
        rs = 1.0 / (stot + 1e-16)
        rs_rep = jnp.dot(rs, r8_ref[...], preferred_element_type=jnp.float32)
        aself_rep = jnp.dot(ex * rs, r8_ref[...],
                            preferred_element_type=jnp.float32)
        o = ((op[0] + op[1]) * rs_rep + aself_rep * h1_ref[...]
             + b1_ref[...])
        eo = jnp.where(o > 0.0, o, jnp.exp(o) - 1.0)
        h2 = jnp.dot(eo, w2_ref[...], preferred_element_type=jnp.float32)
        h2p_ref[...] = jnp.concatenate(
            [h2, jnp.zeros((h2.shape[0], 8), jnp.float32)], axis=1)
        att2_ref[...] = jnp.dot(h2, m2_ref[...],
                                preferred_element_type=jnp.float32)

    g = n // rblk
    return pl.pallas_call(
        body,
        grid=(g,),
        in_specs=[
            pl.BlockSpec((2, rblk, 16), lambda i: (0, i, 0)),
            pl.BlockSpec((2, rblk, 64), lambda i: (0, i, 0)),
            pl.BlockSpec((rblk, 16), lambda i: (i, 0)),
            pl.BlockSpec((rblk, 16), lambda i: (i, 0)),
            pl.BlockSpec((rblk, 64), lambda i: (i, 0)),
            pl.BlockSpec((1, 64), lambda i: (0, 0)),
            pl.BlockSpec((64, 40), lambda i: (0, 0)),
            pl.BlockSpec((40, 8), lambda i: (0, 0)),
            pl.BlockSpec((8, 64), lambda i: (0, 0)),
        ],
        out_specs=[
            pl.BlockSpec((rblk, 48), lambda i: (i, 0)),
            pl.BlockSpec((rblk, 8), lambda i: (i, 0)),
        ],
        out_shape=[
            jax.ShapeDtypeStruct((n, 48), jnp.float32),
            jax.ShapeDtypeStruct((n, 8), jnp.float32),
        ],
    )(sparts, oparts, ts1, td1, h1, b1, w2, m2, r8)


def _tc3(o2parts, s2parts, att2, h2p, b2, n, rblk):
    def body(op_ref, sp_ref, att_ref, h2_ref, b2_ref, out_ref):
        op = op_ref[...]
        att = att_ref[...]
        e = att[:, 0:1] + att[:, 1:2]
        ex = jnp.exp(jnp.where(e > 0.0, e, e * 0.2))
        sp = sp_ref[...]
        stot = sp[:, 0:1] + sp[:, 1:2] + ex
        rs = 1.0 / (stot + 1e-16)
        o = ((op[0][:, 0:40] + op[1][:, 0:40]) * rs
             + (ex * rs) * h2_ref[...][:, 0:40] + b2_ref[...])
        m = jnp.max(o, axis=1, keepdims=True)
        lse = jnp.log(jnp.sum(jnp.exp(o - m), axis=1, keepdims=True)) + m
        out_ref[...] = o - lse

    g = n // rblk
    return pl.pallas_call(
        body,
        grid=(g,),
        in_specs=[
            pl.BlockSpec((2, rblk, 48), lambda i: (0, i, 0)),
            pl.BlockSpec((rblk, 2), lambda i: (i, 0)),
            pl.BlockSpec((rblk, 8), lambda i: (i, 0)),
            pl.BlockSpec((rblk, 48), lambda i: (i, 0)),
            pl.BlockSpec((1, 40), lambda i: (0, 0)),
        ],
        out_specs=pl.BlockSpec((rblk, 40), lambda i: (i, 0)),
        out_shape=jax.ShapeDtypeStruct((n, 40), jnp.float32),
    )(o2parts, s2parts, att2, h2p, b2)


def kernel(x, edge_index, W1, att_src1, att_dst1, b1, W2, att_src2, att_dst2,
           b2):
    n, _ = x.shape
    e_num = edge_index.shape[1]
    h1dim = W1.shape[1]
    c2 = W2.shape[1]
    n_pad = n + 8
    rblk = min(2000, n)

    tot = 2 * math.ceil(e_num / (NW * CH))
    tot += tot % 2
    nch0_l1, nch1_l1 = SPLIT_L1(tot)
    nch0_l2, nch1_l2 = SPLIT_L2(tot)
    max_nch = max(nch0_l1, nch1_l1, nch0_l2, nch1_l2)
    e_pad = NS * tot * CH + max_nch * CH
    pad = e_pad - e_num
    src = jnp.concatenate([edge_index[0], jnp.zeros((pad,), jnp.int32)])
    dst = jnp.concatenate([edge_index[1], jnp.zeros((pad,), jnp.int32)])

    hc = jnp.arange(h1dim)
    ms1 = jnp.zeros((h1dim, 16), jnp.float32).at[hc, hc // 8].set(
        att_src1.reshape(h1dim))
    md1 = jnp.zeros((h1dim, 16), jnp.float32).at[hc, hc // 8].set(
        att_dst1.reshape(h1dim))
    r8 = jnp.zeros((8, h1dim), jnp.float32).at[hc // 8, hc].set(1.0)
    m2 = jnp.zeros((c2, 8), jnp.float32)
    m2 = m2.at[:, 0].set(att_src2.reshape(c2)).at[:, 1].set(
        att_dst2.reshape(c2))

    z16 = jnp.zeros((n_pad, 16), jnp.float32)
    z64 = jnp.zeros((n_pad, 64), jnp.float32)
    z48 = jnp.zeros((n_pad, 48), jnp.float32)
    z1 = jnp.zeros((n_pad,), jnp.float32)

    h1, ts1, td1 = _tc1(x, W1, ms1, md1, n, rblk)
    sparts, oparts = _sc_sweep_l1(n_pad, nch0_l1, nch1_l1)(
        src, dst, ts1, td1, h1, z16, z64)
    h2p, att2 = _tc2(sparts[:, :n, :], oparts[:, :n, :], ts1, td1, h1,
                     b1.reshape(1, -1), W2, m2, r8, n, rblk)

    s2parts, o2parts = _sc_sweep_l2(n_pad, nch0_l2, nch1_l2)(
        src, dst, att2[:, 0], att2[:, 1], h2p, z1, z48)
    return _tc3(o2parts[:, :n, :], s2parts[:, :n].T, att2, h2p,
                b2.reshape(1, -1), n, rblk)


# speedup vs baseline: 1.0938x; 1.0028x over previous
import functools
import math

import jax
import jax.numpy as jnp
from jax import lax
from jax.experimental import pallas as pl
from jax.experimental.pallas import tpu as pltpu
from jax.experimental.pallas import tpu_sc as plsc

NC = 2
NS = 16
NW = NC * NS
LANES = 16
CH = 128


def _split(tot, frac0):
    n0 = int(round(tot * frac0 / 2)) * 2
    n0 = min(max(n0, 2), tot - 2)
    return n0, tot - n0


def SPLIT_L1(tot):
    return _split(tot, 0.525)


def SPLIT_L2(tot):
    return _split(tot, 0.565)


def _mesh():
    return plsc.VectorSubcoreMesh(core_axis_name="c", subcore_axis_name="s")


def _sc_kernel(out_shapes, scratch):
    return functools.partial(
        pl.kernel,
        out_type=tuple(jax.ShapeDtypeStruct(s, jnp.float32)
                       for s in out_shapes),
        mesh=_mesh(),
        compiler_params=pltpu.CompilerParams(use_tc_tiling_on_sc=False),
        scratch_types=scratch,
    )


def _copy_idx(srcall, dstall, srcv, dstv, cur):
    for g in range(CH // LANES):
        sl = pl.ds(cur * CH + g * LANES, LANES)
        so = pl.ds(g * LANES, LANES)
        srcv[so] = srcall[sl]
        dstv[so] = dstall[sl]


def _scatter_idx(srcv, dstv, sidx, trash):
    for g in range(CH // LANES):
        sl = pl.ds(g * LANES, LANES)
        sv = srcv[sl]
        dv = dstv[sl]
        sidx[sl] = jnp.where(sv != dv, dv, trash)


def _sc_sweep_l1(n_pad, nch0, nch1):
    ept = max(nch0, nch1) * CH

    @_sc_kernel([(NC, n_pad, 16), (NC, n_pad, 64)], [
        pltpu.VMEM((ept,), jnp.int32),
        pltpu.VMEM((ept,), jnp.int32),
        pltpu.VMEM((CH,), jnp.int32),
        pltpu.VMEM((CH,), jnp.int32),
        pltpu.VMEM((CH,), jnp.int32),
        pltpu.VMEM((CH,), jnp.int32),
        pltpu.VMEM((CH,), jnp.int32),
        pltpu.VMEM((CH,), jnp.int32),
        pltpu.VMEM((CH, 16), jnp.float32),
        pltpu.VMEM((CH, 16), jnp.float32),
        pltpu.VMEM((CH, 16), jnp.float32),
        pltpu.VMEM((CH, 16), jnp.float32),
        pltpu.VMEM((CH, 64), jnp.float32),
        pltpu.VMEM((CH, 64), jnp.float32),
        pltpu.VMEM_SHARED((n_pad, 16), jnp.float32),
        pltpu.VMEM_SHARED((n_pad, 64), jnp.float32),
        pltpu.SemaphoreType.DMA,
        pltpu.SemaphoreType.DMA,
        pltpu.SemaphoreType.DMA,
        pltpu.SemaphoreType.DMA,
    ])
    def k(srcp, dstp, ts, td, h1, z16, z64, outs, outo, srcall, dstall,
          srcv0, srcv1, dstv0, dstv1, sidx0, sidx1, rs0, rs1, rd0, rd1,
          hr0, hr1, sacc, oacc, gs0, gs1, ss0, ss1):
        c = lax.axis_index("c")
        s = lax.axis_index("s")
        wid = c * NS + s
        trash = n_pad - 8
        lane = lax.iota(jnp.int32, LANES)
        head_idx = [2 * k2 + jnp.right_shift(lane, 3) for k2 in range(4)]
        srcv_ = [srcv0, srcv1]
        dstv_ = [dstv0, dstv1]
        rs_ = [rs0, rs1]
        rd_ = [rd0, rd1]
        hr_ = [hr0, hr1]
        six_ = [sidx0, sidx1]
        gs_ = [gs0, gs1]
        ss_ = [ss0, ss1]

        @pl.when(s == 0)
        def _init():
            pltpu.sync_copy(z16, sacc)
            pltpu.sync_copy(z64, oacc)

        nch_local = jnp.where(c == 0, nch0, nch1)
        base = pl.multiple_of(
            jnp.where(c == 0, s * nch0, 16 * nch0 + s * nch1) * CH, CH)
        pltpu.sync_copy(srcp.at[pl.ds(base, ept)], srcall)
        pltpu.sync_copy(dstp.at[pl.ds(base, ept)], dstall)
        plsc.subcore_barrier()

        def gathers(b):
            return (
                pltpu.make_async_copy(ts.at[srcv_[b]], rs_[b], gs_[b]),
                pltpu.make_async_copy(td.at[dstv_[b]], rd_[b], gs_[b]),
                pltpu.make_async_copy(h1.at[srcv_[b]], hr_[b], gs_[b]),
            )

        def issue(cur, b):
            _copy_idx(srcall, dstall, srcv_[b], dstv_[b], cur)
            for d in gathers(b):
                d.start()

        def scatter_start(b):
            pltpu.async_copy(rs_[b], sacc.at[six_[b]], ss_[b], add=True)
            pltpu.async_copy(hr_[b], oacc.at[six_[b]], ss_[b], add=True)

        def scatter_wait(b):
            pltpu.make_async_copy(rs_[b], sacc.at[six_[b]], ss_[b]).wait()
            pltpu.make_async_copy(hr_[b], oacc.at[six_[b]], ss_[b]).wait()

        issue(0, 0)
        issue(1, 1)

        def gg_body(gg, _):
            for b in range(2):
                cur = 2 * gg + b
                for d in gathers(b):
                    d.wait()
                _scatter_idx(srcv_[b], dstv_[b], six_[b], trash)
                for i in range(CH):
                    e = rs_[b][i] + rd_[b][i]
                    lr = jnp.where(e > 0.0, e, e * 0.2)
                    ex = jnp.exp(lr)
                    rs_[b][i] = ex
                    for k2 in range(4):
                        hsl = pl.ds(k2 * LANES, LANES)
                        av = jnp.take_along_axis(ex, head_idx[k2], axis=0)
                        hr_[b][i, hsl] = hr_[b][i, hsl] * av
                scatter_start(b)

                @pl.when(cur + 2 < nch_local)
                def _next():
                    scatter_wait(b)
                    issue(cur + 2, b)
            return ()

        lax.fori_loop(0, nch_local // 2, gg_body, ())
        scatter_wait(0)
        scatter_wait(1)
        plsc.subcore_barrier()

        @pl.when(s == 0)
        def _flush():
            pltpu.sync_copy(sacc, outs.at[c])
            pltpu.sync_copy(oacc, outo.at[c])

    return k


def _sc_sweep_l2(n_pad, nch0, nch1):
    ept = max(nch0, nch1) * CH

    @_sc_kernel([(NC, n_pad), (NC, n_pad, 48)], [
        pltpu.VMEM((ept,), jnp.int32),
        pltpu.VMEM((ept,), jnp.int32),
        pltpu.VMEM((CH,), jnp.int32),
        pltpu.VMEM((CH,), jnp.int32),
        pltpu.VMEM((CH,), jnp.int32),
        pltpu.VMEM((CH,), jnp.int32),
        pltpu.VMEM((CH,), jnp.int32),
        pltpu.VMEM((CH,), jnp.int32),
        pltpu.VMEM((CH,), jnp.float32),
        pltpu.VMEM((CH,), jnp.float32),
        pltpu.VMEM((CH,), jnp.float32),
        pltpu.VMEM((CH,), jnp.float32),
        pltpu.VMEM((CH, 48), jnp.float32),
        pltpu.VMEM((CH, 48), jnp.float32),
        pltpu.VMEM_SHARED((n_pad,), jnp.float32),
        pltpu.VMEM_SHARED((n_pad, 48), jnp.float32),
        pltpu.SemaphoreType.DMA,
        pltpu.SemaphoreType.DMA,
        pltpu.SemaphoreType.DMA,
        pltpu.SemaphoreType.DMA,
    ])
    def k(srcp, dstp, asrc, adst, h2, z1, z48, outs, outo, srcall, dstall,
          srcv0, srcv1, dstv0, dstv1, sidx0, sidx1, av0, av1, bv0, bv1,
          hr0, hr1, sacc, oacc, gs0, gs1, ss0, ss1):
        c = lax.axis_index("c")
        s = lax.axis_index("s")
        wid = c * NS + s
        trash = n_pad - 8
        lane = lax.iota(jnp.int32, LANES)
        srcv_ = [srcv0, srcv1]
        dstv_ = [dstv0, dstv1]
        av_ = [av0, av1]
        bv_ = [bv0, bv1]
        hr_ = [hr0, hr1]
        six_ = [sidx0, sidx1]
        gs_ = [gs0, gs1]
        ss_ = [ss0, ss1]

        @pl.when(s == 0)
        def _init():
            pltpu.sync_copy(z1, sacc)
            pltpu.sync_copy(z48, oacc)

        nch_local = jnp.where(c == 0, nch0, nch1)
        base = pl.multiple_of(
            jnp.where(c == 0, s * nch0, 16 * nch0 + s * nch1) * CH, CH)
        pltpu.sync_copy(srcp.at[pl.ds(base, ept)], srcall)
        pltpu.sync_copy(dstp.at[pl.ds(base, ept)], dstall)
        plsc.subcore_barrier()

        def gathers(b):
            return (
                pltpu.make_async_copy(asrc.at[srcv_[b]], av_[b], gs_[b]),
                pltpu.make_async_copy(adst.at[dstv_[b]], bv_[b], gs_[b]),
                pltpu.make_async_copy(h2.at[srcv_[b]], hr_[b], gs_[b]),
            )

        def issue(cur, b):
            _copy_idx(srcall, dstall, srcv_[b], dstv_[b], cur)
            for d in gathers(b):
                d.start()

        def scatter_start(b):
            pltpu.async_copy(av_[b], sacc.at[six_[b]], ss_[b], add=True)
            pltpu.async_copy(hr_[b], oacc.at[six_[b]], ss_[b], add=True)

        def scatter_wait(b):
            pltpu.make_async_copy(av_[b], sacc.at[six_[b]], ss_[b]).wait()
            pltpu.make_async_copy(hr_[b], oacc.at[six_[b]], ss_[b]).wait()

        issue(0, 0)
        issue(1, 1)

        def gg_body(gg, _):
            for b in range(2):
                cur = 2 * gg + b
                for d in gathers(b):
                    d.wait()
                _scatter_idx(srcv_[b], dstv_[b], six_[b], trash)
                for g in range(CH // LANES):
                    sl = pl.ds(g * LANES, LANES)
                    e = av_[b][sl] + bv_[b][sl]
                    lr = jnp.where(e > 0.0, e, e * 0.2)
                    av_[b][sl] = jnp.exp(lr)
                for g in range(CH // LANES):
                    a16 = av_[b][pl.ds(g * LANES, LANES)]
                    for j in range(LANES):
                        i = g * LANES + j
                        asp = jnp.take_along_axis(a16, lane * 0 + j, axis=0)
                        for k2 in range(3):
                            hsl = pl.ds(k2 * LANES, LANES)
                            hr_[b][i, hsl] = hr_[b][i, hsl] * asp
                scatter_start(b)

                @pl.when(cur + 2 < nch_local)
                def _next():
                    scatter_wait(b)
                    issue(cur + 2, b)
            return ()

        lax.fori_loop(0, nch_local // 2, gg_body, ())
        scatter_wait(0)
        scatter_wait(1)
        plsc.subcore_barrier()

        @pl.when(s == 0)
        def _flush():
            pltpu.sync_copy(sacc, outs.at[c])
            pltpu.sync_copy(oacc, outo.at[c])

    return k


def _tc1(x, w1, ms1, md1, n, rblk):
    def body(x_ref, w_ref, ms_ref, md_ref, h_ref, ts_ref, td_ref):
        h = jnp.dot(x_ref[...], w_ref[...], preferred_element_type=jnp.float32)
        h_ref[...] = h
        ts_ref[...] = jnp.dot(h, ms_ref[...], preferred_element_type=jnp.float32)
        td_ref[...] = jnp.dot(h, md_ref[...], preferred_element_type=jnp.float32)

    g = n // rblk
    return pl.pallas_call(
        body,
        grid=(g,),
        in_specs=[
            pl.BlockSpec((rblk, 128), lambda i: (i, 0)),
            pl.BlockSpec((128, 64), lambda i: (0, 0)),
            pl.BlockSpec((64, 16), lambda i: (0, 0)),
            pl.BlockSpec((64, 16), lambda i: (0, 0)),
        ],
        out_specs=[
            pl.BlockSpec((rblk, 64), lambda i: (i, 0)),
            pl.BlockSpec((rblk, 16), lambda i: (i, 0)),
            pl.BlockSpec((rblk, 16), lambda i: (i, 0)),
        ],
        out_shape=[
            jax.ShapeDtypeStruct((n, 64), jnp.float32),
            jax.ShapeDtypeStruct((n, 16), jnp.float32),
            jax.ShapeDtypeStruct((n, 16), jnp.float32),
        ],
    )(x, w1, ms1, md1)


def _tc2(sparts, oparts, ts1, td1, h1, b1, w2, m2, r8, n, rblk):
    def body(sp_ref, op_ref, ts_ref, td_ref, h1_ref, b1_ref, w2_ref,
             m2_ref, r8_ref, h2p_ref, att2_ref):
        sp = sp_ref[...]
        op = op_ref[...]
        e = ts_ref[...][:, 0:8] + td_ref[...][:, 0:8]
        ex = jnp.exp(jnp.where(e > 0.0, e, e * 0.2))
        stot = sp[0][:, 0:8] + sp[1][:, 0:8] + ex
        rs = 1.0 / (stot + 1e-16)
        rs_rep = jnp.dot(rs, r8_ref[...], preferred_element_type=jnp.float32)
        aself_rep = jnp.dot(ex * rs, r8_ref[...],
                            preferred_element_type=jnp.float32)
        o = ((op[0] + op[1]) * rs_rep + aself_rep * h1_ref[...]
             + b1_ref[...])
        eo = jnp.where(o > 0.0, o, jnp.exp(o) - 1.0)
        h2 = jnp.dot(eo, w2_ref[...], preferred_element_type=jnp.float32)
        h2p_ref[...] = jnp.concatenate(
            [h2, jnp.zeros((h2.shape[0], 8), jnp.float32)], axis=1)
        att2_ref[...] = jnp.dot(h2, m2_ref[...],
                                preferred_element_type=jnp.float32)

    g = n // rblk
    return pl.pallas_call(
        body,
        grid=(g,),
        in_specs=[
            pl.BlockSpec((2, rblk, 16), lambda i: (0, i, 0)),
            pl.BlockSpec((2, rblk, 64), lambda i: (0, i, 0)),
            pl.BlockSpec((rblk, 16), lambda i: (i, 0)),
            pl.BlockSpec((rblk, 16), lambda i: (i, 0)),
            pl.BlockSpec((rblk, 64), lambda i: (i, 0)),
            pl.BlockSpec((1, 64), lambda i: (0, 0)),
            pl.BlockSpec((64, 40), lambda i: (0, 0)),
            pl.BlockSpec((40, 8), lambda i: (0, 0)),
            pl.BlockSpec((8, 64), lambda i: (0, 0)),
        ],
        out_specs=[
            pl.BlockSpec((rblk, 48), lambda i: (i, 0)),
            pl.BlockSpec((rblk, 8), lambda i: (i, 0)),
        ],
        out_shape=[
            jax.ShapeDtypeStruct((n, 48), jnp.float32),
            jax.ShapeDtypeStruct((n, 8), jnp.float32),
        ],
    )(sparts, oparts, ts1, td1, h1, b1, w2, m2, r8)


def _tc3(o2parts, s2parts, att2, h2p, b2, n, rblk):
    def body(op_ref, sp_ref, att_ref, h2_ref, b2_ref, out_ref):
        op = op_ref[...]
        att = att_ref[...]
        e = att[:, 0:1] + att[:, 1:2]
        ex = jnp.exp(jnp.where(e > 0.0, e, e * 0.2))
        sp = sp_ref[...]
        stot = sp[:, 0:1] + sp[:, 1:2] + ex
        rs = 1.0 / (stot + 1e-16)
        o = ((op[0][:, 0:40] + op[1][:, 0:40]) * rs
             + (ex * rs) * h2_ref[...][:, 0:40] + b2_ref[...])
        m = jnp.max(o, axis=1, keepdims=True)
        lse = jnp.log(jnp.sum(jnp.exp(o - m), axis=1, keepdims=True)) + m
        out_ref[...] = o - lse

    g = n // rblk
    return pl.pallas_call(
        body,
        grid=(g,),
        in_specs=[
            pl.BlockSpec((2, rblk, 48), lambda i: (0, i, 0)),
            pl.BlockSpec((rblk, 2), lambda i: (i, 0)),
            pl.BlockSpec((rblk, 8), lambda i: (i, 0)),
            pl.BlockSpec((rblk, 48), lambda i: (i, 0)),
            pl.BlockSpec((1, 40), lambda i: (0, 0)),
        ],
        out_specs=pl.BlockSpec((rblk, 40), lambda i: (i, 0)),
        out_shape=jax.ShapeDtypeStruct((n, 40), jnp.float32),
    )(o2parts, s2parts, att2, h2p, b2)


def kernel(x, edge_index, W1, att_src1, att_dst1, b1, W2, att_src2, att_dst2,
           b2):
    n, _ = x.shape
    e_num = edge_index.shape[1]
    h1dim = W1.shape[1]
    c2 = W2.shape[1]
    n_pad = n + 8
    rblk = min(2000, n)

    tot = 2 * math.ceil(e_num / (NW * CH))
    tot += tot % 2
    nch0_l1, nch1_l1 = SPLIT_L1(tot)
    nch0_l2, nch1_l2 = SPLIT_L2(tot)
    max_nch = max(nch0_l1, nch1_l1, nch0_l2, nch1_l2)
    e_pad = NS * tot * CH + max_nch * CH
    pad = e_pad - e_num
    src = jnp.concatenate([edge_index[0], jnp.zeros((pad,), jnp.int32)])
    dst = jnp.concatenate([edge_index[1], jnp.zeros((pad,), jnp.int32)])

    hc = jnp.arange(h1dim)
    ms1 = jnp.zeros((h1dim, 16), jnp.float32).at[hc, hc // 8].set(
        att_src1.reshape(h1dim))
    md1 = jnp.zeros((h1dim, 16), jnp.float32).at[hc, hc // 8].set(
        att_dst1.reshape(h1dim))
    r8 = jnp.zeros((8, h1dim), jnp.float32).at[hc // 8, hc].set(1.0)
    m2 = jnp.zeros((c2, 8), jnp.float32)
    m2 = m2.at[:, 0].set(att_src2.reshape(c2)).at[:, 1].set(
        att_dst2.reshape(c2))

    z16 = jnp.zeros((n_pad, 16), jnp.float32)
    z64 = jnp.zeros((n_pad, 64), jnp.float32)
    z48 = jnp.zeros((n_pad, 48), jnp.float32)
    z1 = jnp.zeros((n_pad,), jnp.float32)

    h1, ts1, td1 = _tc1(x, W1, ms1, md1, n, rblk)
    sparts, oparts = _sc_sweep_l1(n_pad, nch0_l1, nch1_l1)(
        src, dst, ts1, td1, h1, z16, z64)
    h2p, att2 = _tc2(sparts[:, :n, :], oparts[:, :n, :], ts1, td1, h1,
                     b1.reshape(1, -1), W2, m2, r8, n, rblk)

    s2parts, o2parts = _sc_sweep_l2(n_pad, nch0_l2, nch1_l2)(
        src, dst, att2[:, 0], att2[:, 1], h2p, z1, z48)
    return _tc3(o2parts[:, :n, :], s2parts[:, :n].T, att2, h2p,
                b2.reshape(1, -1), n, rblk)
